# baseline (speedup 1.0000x reference)
import functools

import jax
import jax.numpy as jnp
from jax.experimental import pallas as pl
from jax.experimental.pallas import tpu as pltpu

K5 = 5
PAD = 2
ROFF = 32
COFF = 64


def _iota2(r, c):
    return (jax.lax.broadcasted_iota(jnp.int32, (r, c), 0),
            jax.lax.broadcasted_iota(jnp.int32, (r, c), 1))


def _build_sels(h2, w, mw, wo):
    rbo = h2 // 2
    i, j = _iota2(2 * ROFF, h2)
    srow = ((i < rbo) & (j == 2 * i)) | \
           ((i >= ROFF) & (i < ROFF + rbo) & (j == 2 * (i - ROFF) + 1))
    i, j = _iota2(w, 128)
    c1 = ((j < mw) & (i == 2 * (j - PAD))) | \
         ((j >= COFF) & (j < COFF + mw) & (i == 2 * (j - COFF - PAD) + 1))
    i, j = _iota2(2 * mw, 128)
    ev = jnp.where(j < wo, 2 * j, 2 * (j - wo) + mw)
    od = jnp.where(j - COFF < wo, 2 * (j - COFF) + 1, 2 * (j - COFF - wo) + mw + 1)
    c2 = ((j < 2 * wo) & (i == ev)) | \
         ((j >= COFF) & (j < COFF + 2 * wo) & (i == od))
    return (srow.astype(jnp.float32), c1.astype(jnp.float32),
            c2.astype(jnp.float32))


def _convnet_kernel(x_ref, w1_ref, b1_ref, w2_ref, b2_ref, srow_ref, c1_ref,
                    c2_ref, o_ref, xp5, mid5, act8, *, cin, c1, c2, h, w):
    h2, w2d = h // 2, w // 2
    ho, wo = h // 4, w // 4
    mw = w2d + 2 * PAD
    rb = h // 2
    rbo = rb // 2

    xp5[...] = jnp.zeros_like(xp5)
    mid5[...] = jnp.zeros_like(mid5)

    def pool(act, csel, n_in, n_out):
        r = jnp.dot(srow_ref[:, 0:act.shape[0]], act,
                    preferred_element_type=jnp.float32)
        rows = jnp.maximum(r[0:act.shape[0] // 2], r[ROFF:ROFF + act.shape[0] // 2])
        p = jnp.dot(rows, csel, preferred_element_type=jnp.float32)
        return jnp.maximum(p[:, 0:n_out], p[:, COFF:COFF + n_out])

    for im in range(2):
        for dw in range(K5):
            lo = max(0, PAD - dw)
            hi = min(w, w + PAD - dw)
            xp5[dw, :, PAD:PAD + h, lo:hi] = x_ref[im, :, :, lo + dw - PAD:hi + dw - PAD]

        for s in range(h // rb):
            r0 = s * rb
            for g in range(2):
                accs = [None] * 4
                for ci in range(cin):
                    for dh in range(K5):
                        for dw in range(K5):
                            tap = xp5[dw, ci, r0 + dh:r0 + dh + rb, 0:w]
                            for c in range(4):
                                co = 4 * g + c
                                wv = w1_ref[((co * cin + ci) * K5 + dh) * K5 + dw]
                                t = tap * wv
                                accs[c] = t if accs[c] is None else accs[c] + t
                for c in range(4):
                    co = 4 * g + c
                    act8[co, 0:rb, 0:w] = jnp.maximum(accs[c] + b1_ref[co], 0.0)
            for co in range(c1):
                pooled = pool(act8[co, 0:rb, 0:w], c1_ref[...], w, mw)
                mid5[0, co, PAD + s * rbo:PAD + (s + 1) * rbo,
                     im * mw:(im + 1) * mw] = pooled

    for dw in range(1, K5):
        mid5[dw, :, :, 0:2 * mw - dw] = mid5[0, :, :, dw:2 * mw]

    for g in range(2):
        accs = [None] * 4
        for ci in range(c1):
            for dh in range(K5):
                for dw in range(K5):
                    tap = mid5[dw, ci, dh:dh + h2, 0:2 * mw]
                    for c in range(4):
                        co = 4 * g + c
                        wv = w2_ref[((co * c1 + ci) * K5 + dh) * K5 + dw]
                        t = tap * wv
                        accs[c] = t if accs[c] is None else accs[c] + t
        for c in range(4):
            co = 4 * g + c
            act8[co, 0:h2, 0:2 * mw] = jnp.maximum(accs[c] + b2_ref[co], 0.0)
    for co in range(c2):
        pooled = pool(act8[co, 0:h2, 0:2 * mw], c2_ref[...], 2 * mw, 2 * wo)
        o_ref[0, co, :, :] = pooled[:, 0:wo]
        o_ref[1, co, :, :] = pooled[:, wo:2 * wo]


def _conv_layers(x, w1, b1, w2, b2):
    n, cin, h, w = x.shape
    c1 = w1.shape[0]
    c2 = w2.shape[0]
    assert n % 2 == 0 and h % 4 == 0 and w % 4 == 0, (n, h, w)
    srow, c1sel, c2sel = _build_sels(h // 2, w, w // 2 + 2 * PAD, w // 4)

    kern = functools.partial(_convnet_kernel, cin=cin, c1=c1, c2=c2, h=h, w=w)
    return pl.pallas_call(
        kern,
        out_shape=jax.ShapeDtypeStruct((n, c2, h // 4, w // 4), jnp.float32),
        grid=(n // 2,),
        in_specs=[
            pl.BlockSpec((2, cin, h, w), lambda i: (i, 0, 0, 0)),
            pl.BlockSpec(memory_space=pltpu.MemorySpace.SMEM),
            pl.BlockSpec(memory_space=pltpu.MemorySpace.SMEM),
            pl.BlockSpec(memory_space=pltpu.MemorySpace.SMEM),
            pl.BlockSpec(memory_space=pltpu.MemorySpace.SMEM),
            pl.BlockSpec(srow.shape, lambda i: (0, 0)),
            pl.BlockSpec(c1sel.shape, lambda i: (0, 0)),
            pl.BlockSpec(c2sel.shape, lambda i: (0, 0)),
        ],
        out_specs=pl.BlockSpec((2, c2, h // 4, w // 4), lambda i: (i, 0, 0, 0)),
        scratch_shapes=[
            pltpu.VMEM((K5, cin, h + 4 * PAD, 128), jnp.float32),
            pltpu.VMEM((K5, c1, h // 2 + 4 * PAD, 128), jnp.float32),
            pltpu.VMEM((c1, 64, 128), jnp.float32),
        ],
        compiler_params=pltpu.CompilerParams(dimension_semantics=("parallel",)),
    )(x,
      w1.reshape(-1).astype(jnp.float32), b1.astype(jnp.float32),
      w2.reshape(-1).astype(jnp.float32), b2.astype(jnp.float32),
      srow, c1sel, c2sel)


def _fc_kernel(a_ref, w_ref, b_ref, o_ref):
    o_ref[...] = (jnp.dot(a_ref[...], w_ref[...],
                          preferred_element_type=jnp.float32) + b_ref[...])


def _fc(a, w_t, b):
    m, k = a.shape
    k2, nf = w_t.shape
    assert k == k2
    return pl.pallas_call(
        _fc_kernel,
        out_shape=jax.ShapeDtypeStruct((m, nf), jnp.float32),
        grid=(2,),
        in_specs=[pl.BlockSpec((m // 2, k), lambda i: (i, 0)),
                  pl.BlockSpec((k, nf), lambda i: (0, 0)),
                  pl.BlockSpec((1, nf), lambda i: (0, 0))],
        out_specs=pl.BlockSpec((m // 2, nf), lambda i: (i, 0)),
        compiler_params=pltpu.CompilerParams(dimension_semantics=("parallel",)),
    )(a, w_t, b.reshape(1, nf))


def kernel(x, w1, b1, w2, b2, fc_wt, fc_b):
    y = _conv_layers(x, w1, b1, w2, b2)
    flat = y.reshape(y.shape[0], -1)
    return _fc(flat, fc_wt, fc_b)


# speedup vs baseline: 1.0583x; 1.0583x over previous
import functools

import jax
import jax.numpy as jnp
from jax.experimental import pallas as pl
from jax.experimental.pallas import tpu as pltpu

K5 = 5
PAD = 2
ROFF = 32
COFF = 64


def _iota2(r, c):
    return (jax.lax.broadcasted_iota(jnp.int32, (r, c), 0),
            jax.lax.broadcasted_iota(jnp.int32, (r, c), 1))


def _build_sels(h2, w, mw, wo):
    rbo = h2 // 2
    i, j = _iota2(2 * ROFF, h2)
    srow = ((i < rbo) & (j == 2 * i)) | \
           ((i >= ROFF) & (i < ROFF + rbo) & (j == 2 * (i - ROFF) + 1))
    i, j = _iota2(w, 128)
    c1 = ((j < mw) & (i == 2 * (j - PAD))) | \
         ((j >= COFF) & (j < COFF + mw) & (i == 2 * (j - COFF - PAD) + 1))
    i, j = _iota2(2 * mw, 128)
    ev = jnp.where(j < wo, 2 * j, 2 * (j - wo) + mw)
    od = jnp.where(j - COFF < wo, 2 * (j - COFF) + 1, 2 * (j - COFF - wo) + mw + 1)
    c2 = ((j < 2 * wo) & (i == ev)) | \
         ((j >= COFF) & (j < COFF + 2 * wo) & (i == od))
    return (srow.astype(jnp.float32), c1.astype(jnp.float32),
            c2.astype(jnp.float32))


def _convnet_kernel(x_ref, w1_ref, b1_ref, w2_ref, b2_ref, srow_ref, c1_ref,
                    c2_ref, o_ref, xp5, mid5, act8, *, cin, c1, c2, h, w):
    h2, w2d = h // 2, w // 2
    ho, wo = h // 4, w // 4
    mw = w2d + 2 * PAD
    rb = h // 2
    rbo = rb // 2

    xp5[...] = jnp.zeros_like(xp5)
    mid5[...] = jnp.zeros_like(mid5)

    def pool(act, csel, n_out):
        sr = act.shape[0]
        r = jnp.dot(srow_ref[:, 0:sr], act, preferred_element_type=jnp.float32)
        rows = jnp.maximum(r[0:sr // 2], r[ROFF:ROFF + sr // 2])
        p = jnp.dot(rows, csel, preferred_element_type=jnp.float32)
        return jnp.maximum(p[:, 0:n_out], p[:, COFF:COFF + n_out])


    for im in range(2):
        for dw in range(K5):
            lo = max(0, PAD - dw)
            hi = min(w, w + PAD - dw)
            xp5[dw, :, PAD:PAD + h, lo:hi] = x_ref[im, :, :, lo + dw - PAD:hi + dw - PAD]

        for s in range(h // rb):
            r0 = s * rb

            for g in range(2):
                accs = [None] * 4
                for ci in range(cin):
                    for dh in range(K5):
                        for dw in range(K5):
                            tap = xp5[dw, ci, r0 + dh:r0 + dh + rb, 0:w]
                            for c in range(4):
                                co = 4 * g + c
                                wv = w1_ref[((co * cin + ci) * K5 + dh) * K5 + dw, 0:w]
                                t = tap * wv
                                accs[c] = t if accs[c] is None else accs[c] + t
                for c in range(4):
                    co = 4 * g + c
                    act8[co, 0:rb, 0:w] = jnp.maximum(accs[c] + b1_ref[co], 0.0)
            for co in range(c1):
                pooled = pool(act8[co, 0:rb, 0:w], c1_ref[...], mw)
                mid5[0, co, PAD + s * rbo:PAD + (s + 1) * rbo,
                     im * mw:(im + 1) * mw] = pooled

    for dw in range(1, K5):
        mid5[dw, :, :, 0:2 * mw - dw] = mid5[0, :, :, dw:2 * mw]

    for g in range(2):
        accs = [None] * 4
        for ci in range(c1):
            for dh in range(K5):
                for dw in range(K5):
                    tap = mid5[dw, ci, dh:dh + h2, 0:2 * mw]
                    for c in range(4):
                        co = 4 * g + c
                        wv = w2_ref[((co * c1 + ci) * K5 + dh) * K5 + dw, 0:2 * mw]
                        t = tap * wv
                        accs[c] = t if accs[c] is None else accs[c] + t
        for c in range(4):
            co = 4 * g + c
            act8[co, 0:h2, 0:2 * mw] = jnp.maximum(accs[c] + b2_ref[co], 0.0)
    for co in range(c2):
        pooled = pool(act8[co, 0:h2, 0:2 * mw], c2_ref[...], 2 * wo)
        o_ref[0, co, :, :] = pooled[:, 0:wo]
        o_ref[1, co, :, :] = pooled[:, wo:2 * wo]


def _conv_layers(x, w1, b1, w2, b2):
    n, cin, h, w = x.shape
    c1 = w1.shape[0]
    c2 = w2.shape[0]
    assert n % 2 == 0 and h % 4 == 0 and w % 4 == 0, (n, h, w)
    srow, c1sel, c2sel = _build_sels(h // 2, w, w // 2 + 2 * PAD, w // 4)

    kern = functools.partial(_convnet_kernel, cin=cin, c1=c1, c2=c2, h=h, w=w)
    return pl.pallas_call(
        kern,
        out_shape=jax.ShapeDtypeStruct((n, c2, h // 4, w // 4), jnp.float32),
        grid=(n // 2,),
        in_specs=[
            pl.BlockSpec((2, cin, h, w), lambda i: (i, 0, 0, 0)),
            pl.BlockSpec((c1 * cin * K5 * K5, 128), lambda i: (0, 0)),
            pl.BlockSpec(memory_space=pltpu.MemorySpace.SMEM),
            pl.BlockSpec((c2 * c1 * K5 * K5, 128), lambda i: (0, 0)),
            pl.BlockSpec(memory_space=pltpu.MemorySpace.SMEM),
            pl.BlockSpec(srow.shape, lambda i: (0, 0)),
            pl.BlockSpec(c1sel.shape, lambda i: (0, 0)),
            pl.BlockSpec(c2sel.shape, lambda i: (0, 0)),
        ],
        out_specs=pl.BlockSpec((2, c2, h // 4, w // 4), lambda i: (i, 0, 0, 0)),
        scratch_shapes=[
            pltpu.VMEM((K5, cin, h + 4 * PAD, 128), jnp.float32),
            pltpu.VMEM((K5, c1, h // 2 + 4 * PAD, 128), jnp.float32),
            pltpu.VMEM((c1, 64, 128), jnp.float32),
        ],
        compiler_params=pltpu.CompilerParams(dimension_semantics=("parallel",)),
    )(x,
      jnp.broadcast_to(w1.reshape(-1).astype(jnp.float32)[:, None],
                       (c1 * cin * K5 * K5, 128)),
      b1.astype(jnp.float32),
      jnp.broadcast_to(w2.reshape(-1).astype(jnp.float32)[:, None],
                       (c2 * c1 * K5 * K5, 128)),
      b2.astype(jnp.float32),
      srow, c1sel, c2sel)


def _fc_kernel(a_ref, w_ref, b_ref, o_ref):
    o_ref[...] = (jnp.dot(a_ref[...], w_ref[...],
                          preferred_element_type=jnp.float32) + b_ref[...])


def _fc(a, w_t, b):
    m, k = a.shape
    k2, nf = w_t.shape
    assert k == k2
    return pl.pallas_call(
        _fc_kernel,
        out_shape=jax.ShapeDtypeStruct((m, nf), jnp.float32),
        grid=(2,),
        in_specs=[pl.BlockSpec((m // 2, k), lambda i: (i, 0)),
                  pl.BlockSpec((k, nf), lambda i: (0, 0)),
                  pl.BlockSpec((1, nf), lambda i: (0, 0))],
        out_specs=pl.BlockSpec((m // 2, nf), lambda i: (i, 0)),
        compiler_params=pltpu.CompilerParams(dimension_semantics=("parallel",)),
    )(a, w_t, b.reshape(1, nf))


def kernel(x, w1, b1, w2, b2, fc_wt, fc_b):
    y = _conv_layers(x, w1, b1, w2, b2)
    flat = y.reshape(y.shape[0], -1)
    return _fc(flat, fc_wt, fc_b)
